# Initial kernel scaffold; baseline (speedup 1.0000x reference)
import jax
import jax.numpy as jnp
from jax.experimental import pallas as pl


def kernel(x, We1, be1, We2, be2, Wd1, bd1, Wd2, bd2, codebooks):
    raise NotImplementedError("write your pallas kernel here")


# speedup vs baseline: 1.0860x; 1.0860x over previous
import functools

import jax
import jax.numpy as jnp
from jax.experimental import pallas as pl
from jax.experimental.pallas import tpu as pltpu

FRAME_DIM = 256
CHUNK_LEN = 32
LATENT_DIM = 256
CODEBOOK_SIZE = 1024
LEVELS = 8
HIDDEN = 1024
BETA = 0.25
USAGE_REG = 0.001
BATCH = 1024

TILE = 128
NB = BATCH // TILE
IN_DIM = CHUNK_LEN * FRAME_DIM


def _enc_vq_kernel(xf_ref, We1_ref, be1_ref, We2_ref, be2_ref, cb_ref,
                   zq_ref, codes_ref, qloss_ref, usage_ref, counts_ref):
    i = pl.program_id(0)

    @pl.when(i == 0)
    def _init():
        qloss_ref[...] = jnp.zeros((1, 1), jnp.float32)
        usage_ref[...] = jnp.zeros((1, 1), jnp.float32)
        counts_ref[...] = jnp.zeros_like(counts_ref)

    h = jnp.maximum(jnp.dot(xf_ref[...], We1_ref[...],
                            preferred_element_type=jnp.float32)
                    + be1_ref[0, :], 0.0)
    z = jnp.dot(h, We2_ref[...], preferred_element_type=jnp.float32) \
        + be2_ref[0, :]

    residual = z
    z_q = jnp.zeros_like(z)
    q_acc = jnp.float32(0.0)
    idx_rows = []
    count_rows = []
    col_iota = jax.lax.broadcasted_iota(jnp.int32, (TILE, CODEBOOK_SIZE), 1)
    for l in range(LEVELS):
        cb = cb_ref[l]
        cb2 = jnp.sum(cb * cb, axis=-1)
        rr = jnp.sum(residual * residual, axis=-1, keepdims=True)
        cross = jnp.dot(residual, cb.T, preferred_element_type=jnp.float32)
        dists = rr - 2.0 * cross + cb2[None, :]
        m = jnp.min(dists, axis=-1, keepdims=True)
        idx = jnp.min(jnp.where(dists == m, col_iota, CODEBOOK_SIZE),
                      axis=-1)
        onehot = (col_iota == idx[:, None]).astype(jnp.float32)
        q = jnp.dot(onehot, cb, preferred_element_type=jnp.float32,
                    precision=jax.lax.Precision.HIGHEST)
        idx_rows.append(idx)
        count_rows.append(jnp.sum(onehot, axis=0))
        q_acc = q_acc + jnp.sum((residual - q) ** 2)
        z_q = z_q + q
        residual = residual - q

    zq_ref[...] = z_q
    codes_ref[...] = jnp.stack(idx_rows, axis=0)
    counts_ref[...] += jnp.stack(count_rows, axis=0)
    qloss_ref[...] += ((1.0 + BETA) / (BATCH * LATENT_DIM)
                       * q_acc).reshape(1, 1)

    @pl.when(i == NB - 1)
    def _finalize():
        p = counts_ref[...] * (1.0 / BATCH)
        usage_ref[...] = (USAGE_REG
                          * jnp.sum(p * jnp.log(p + 1e-10))).reshape(1, 1)


def _dec_kernel(zq_ref, Wd1_ref, bd1_ref, Wd2_ref, bd2_ref, out_ref):
    hd = jnp.maximum(jnp.dot(zq_ref[...], Wd1_ref[...],
                             preferred_element_type=jnp.float32)
                     + bd1_ref[0, :], 0.0)
    out_ref[...] = jnp.dot(hd, Wd2_ref[...],
                           preferred_element_type=jnp.float32) + bd2_ref[0, :]


@jax.jit
def kernel(x, We1, be1, We2, be2, Wd1, bd1, Wd2, bd2, codebooks):
    B = x.shape[0]
    xf = x.reshape(B, IN_DIM)

    z_q, codes_lm, q_loss, usage_loss = pl.pallas_call(
        _enc_vq_kernel,
        grid=(NB,),
        in_specs=[
            pl.BlockSpec((TILE, IN_DIM), lambda i: (i, 0)),
            pl.BlockSpec((IN_DIM, HIDDEN), lambda i: (0, 0)),
            pl.BlockSpec((1, HIDDEN), lambda i: (0, 0)),
            pl.BlockSpec((HIDDEN, LATENT_DIM), lambda i: (0, 0)),
            pl.BlockSpec((1, LATENT_DIM), lambda i: (0, 0)),
            pl.BlockSpec((LEVELS, CODEBOOK_SIZE, LATENT_DIM),
                         lambda i: (0, 0, 0)),
        ],
        out_specs=[
            pl.BlockSpec((TILE, LATENT_DIM), lambda i: (i, 0)),
            pl.BlockSpec((LEVELS, TILE), lambda i: (0, i)),
            pl.BlockSpec((1, 1), lambda i: (0, 0)),
            pl.BlockSpec((1, 1), lambda i: (0, 0)),
        ],
        out_shape=[
            jax.ShapeDtypeStruct((B, LATENT_DIM), jnp.float32),
            jax.ShapeDtypeStruct((LEVELS, B), jnp.int32),
            jax.ShapeDtypeStruct((1, 1), jnp.float32),
            jax.ShapeDtypeStruct((1, 1), jnp.float32),
        ],
        scratch_shapes=[pltpu.VMEM((LEVELS, CODEBOOK_SIZE), jnp.float32)],
    )(xf, We1, be1.reshape(1, HIDDEN), We2, be2.reshape(1, LATENT_DIM),
      codebooks)

    recon_flat = pl.pallas_call(
        _dec_kernel,
        grid=(NB,),
        in_specs=[
            pl.BlockSpec((TILE, LATENT_DIM), lambda i: (i, 0)),
            pl.BlockSpec((LATENT_DIM, HIDDEN), lambda i: (0, 0)),
            pl.BlockSpec((1, HIDDEN), lambda i: (0, 0)),
            pl.BlockSpec((HIDDEN, IN_DIM), lambda i: (0, 0)),
            pl.BlockSpec((1, IN_DIM), lambda i: (0, 0)),
        ],
        out_specs=pl.BlockSpec((TILE, IN_DIM), lambda i: (i, 0)),
        out_shape=jax.ShapeDtypeStruct((B, IN_DIM), jnp.float32),
    )(z_q, Wd1, bd1.reshape(1, HIDDEN), Wd2, bd2.reshape(1, IN_DIM))

    recon = recon_flat.reshape(B, CHUNK_LEN, FRAME_DIM)
    codes = codes_lm.T
    return recon, codes, q_loss.reshape(()), usage_loss.reshape(()), z_q


# speedup vs baseline: 1.6728x; 1.5403x over previous
import jax
import jax.numpy as jnp
from jax.experimental import pallas as pl
from jax.experimental.pallas import tpu as pltpu

FRAME_DIM = 256
CHUNK_LEN = 32
LATENT_DIM = 256
CODEBOOK_SIZE = 1024
LEVELS = 8
HIDDEN = 1024
BETA = 0.25
USAGE_REG = 0.001
BATCH = 1024

TILE = 256
NB = BATCH // TILE
IN_DIM = CHUNK_LEN * FRAME_DIM


def _enc_kernel(x_ref, We1_ref, be1_ref, We2_ref, be2_ref, z_ref):
    xf = x_ref[...].reshape(TILE, IN_DIM)
    h = jnp.maximum(jnp.dot(xf, We1_ref[...],
                            preferred_element_type=jnp.float32)
                    + be1_ref[0, :], 0.0)
    z_ref[...] = jnp.dot(h, We2_ref[...],
                         preferred_element_type=jnp.float32) + be2_ref[0, :]


def _vq_kernel(z_ref, cb_ref, zq_ref, codes_ref, qloss_ref, usage_ref):
    z = z_ref[...]
    residual = z
    z_q = jnp.zeros_like(z)
    q_acc = jnp.float32(0.0)
    idx_rows = []
    count_rows = []
    fiota = jax.lax.broadcasted_iota(
        jnp.int32, (BATCH, CODEBOOK_SIZE), 1).astype(jnp.float32)
    for l in range(LEVELS):
        cb = cb_ref[l]
        cb2 = jnp.sum(cb * cb, axis=-1)
        rr = jnp.sum(residual * residual, axis=-1, keepdims=True)
        cross = jnp.dot(-2.0 * residual, cb.T,
                        preferred_element_type=jnp.float32)
        dists = rr + cross + cb2[None, :]
        m = jnp.min(dists, axis=-1, keepdims=True)
        idxf = jnp.min(jnp.where(dists == m, fiota,
                                 jnp.float32(CODEBOOK_SIZE)), axis=-1)
        onehot = (fiota == idxf[:, None]).astype(jnp.float32)
        q = jnp.dot(onehot, cb, preferred_element_type=jnp.float32,
                    precision=jax.lax.Precision.HIGHEST)
        idx_rows.append(idxf.astype(jnp.int32))
        count_rows.append(jnp.sum(onehot, axis=0))
        q_acc = q_acc + jnp.sum((residual - q) ** 2)
        z_q = z_q + q
        residual = residual - q

    zq_ref[...] = z_q
    codes_ref[...] = jnp.stack(idx_rows, axis=0).T
    qloss_ref[...] = ((1.0 + BETA) / (BATCH * LATENT_DIM)
                      * q_acc).reshape(1, 1)
    p = jnp.stack(count_rows, axis=0) * (1.0 / BATCH)
    usage_ref[...] = (USAGE_REG
                      * jnp.sum(p * jnp.log(p + 1e-10))).reshape(1, 1)


def _dec_kernel(zq_ref, Wd1_ref, bd1_ref, Wd2_ref, bd2_ref, out_ref):
    hd = jnp.maximum(jnp.dot(zq_ref[...], Wd1_ref[...],
                             preferred_element_type=jnp.float32)
                     + bd1_ref[0, :], 0.0)
    r = jnp.dot(hd, Wd2_ref[...],
                preferred_element_type=jnp.float32) + bd2_ref[0, :]
    out_ref[...] = r.reshape(TILE, CHUNK_LEN, FRAME_DIM)


@jax.jit
def kernel(x, We1, be1, We2, be2, Wd1, bd1, Wd2, bd2, codebooks):
    B = x.shape[0]

    z = pl.pallas_call(
        _enc_kernel,
        grid=(NB,),
        in_specs=[
            pl.BlockSpec((TILE, CHUNK_LEN, FRAME_DIM), lambda i: (i, 0, 0)),
            pl.BlockSpec((IN_DIM, HIDDEN), lambda i: (0, 0)),
            pl.BlockSpec((1, HIDDEN), lambda i: (0, 0)),
            pl.BlockSpec((HIDDEN, LATENT_DIM), lambda i: (0, 0)),
            pl.BlockSpec((1, LATENT_DIM), lambda i: (0, 0)),
        ],
        out_specs=pl.BlockSpec((TILE, LATENT_DIM), lambda i: (i, 0)),
        out_shape=jax.ShapeDtypeStruct((B, LATENT_DIM), jnp.float32),
    )(x, We1, be1.reshape(1, HIDDEN), We2, be2.reshape(1, LATENT_DIM))

    z_q, codes, q_loss, usage_loss = pl.pallas_call(
        _vq_kernel,
        out_shape=[
            jax.ShapeDtypeStruct((B, LATENT_DIM), jnp.float32),
            jax.ShapeDtypeStruct((B, LEVELS), jnp.int32),
            jax.ShapeDtypeStruct((1, 1), jnp.float32),
            jax.ShapeDtypeStruct((1, 1), jnp.float32),
        ],
    )(z, codebooks)

    recon = pl.pallas_call(
        _dec_kernel,
        grid=(NB,),
        in_specs=[
            pl.BlockSpec((TILE, LATENT_DIM), lambda i: (i, 0)),
            pl.BlockSpec((LATENT_DIM, HIDDEN), lambda i: (0, 0)),
            pl.BlockSpec((1, HIDDEN), lambda i: (0, 0)),
            pl.BlockSpec((HIDDEN, IN_DIM), lambda i: (0, 0)),
            pl.BlockSpec((1, IN_DIM), lambda i: (0, 0)),
        ],
        out_specs=pl.BlockSpec((TILE, CHUNK_LEN, FRAME_DIM),
                               lambda i: (i, 0, 0)),
        out_shape=jax.ShapeDtypeStruct((B, CHUNK_LEN, FRAME_DIM),
                                       jnp.float32),
    )(z_q, Wd1, bd1.reshape(1, HIDDEN), Wd2, bd2.reshape(1, IN_DIM))

    return recon, codes, q_loss.reshape(()), usage_loss.reshape(()), z_q


# speedup vs baseline: 2.0374x; 1.2180x over previous
import jax
import jax.numpy as jnp
from jax.experimental import pallas as pl
from jax.experimental.pallas import tpu as pltpu

FRAME_DIM = 256
CHUNK_LEN = 32
LATENT_DIM = 256
CODEBOOK_SIZE = 1024
LEVELS = 8
HIDDEN = 1024
BETA = 0.25
USAGE_REG = 0.001
BATCH = 1024

TILE = 256
NB = BATCH // TILE
IN_DIM = CHUNK_LEN * FRAME_DIM


def _enc_kernel(x_ref, We1_ref, be1_ref, We2_ref, be2_ref, z_ref):
    xf = x_ref[...].reshape(TILE, IN_DIM)
    h = jnp.maximum(jnp.dot(xf, We1_ref[...],
                            preferred_element_type=jnp.float32)
                    + be1_ref[0, :], 0.0)
    z_ref[...] = jnp.dot(h, We2_ref[...],
                         preferred_element_type=jnp.float32) + be2_ref[0, :]


def _vq_kernel(z_ref, cb_ref, zq_ref, codes_ref, qloss_ref, usage_ref):
    z = z_ref[...]
    residual = z
    z_q = jnp.zeros_like(z)
    q_acc = jnp.float32(0.0)
    idx_rows = []
    count_rows = []
    fiota = jax.lax.broadcasted_iota(
        jnp.int32, (BATCH, CODEBOOK_SIZE), 1).astype(jnp.float32)
    ones_row = jnp.ones((1, BATCH), jnp.bfloat16)
    for l in range(LEVELS):
        cb = cb_ref[l]
        cb_hi = cb.astype(jnp.bfloat16)
        r1 = cb - cb_hi.astype(jnp.float32)
        cb_mid = r1.astype(jnp.bfloat16)
        cb_lo = (r1 - cb_mid.astype(jnp.float32)).astype(jnp.bfloat16)
        cb2 = jnp.sum(cb * cb, axis=-1)
        rr = jnp.sum(residual * residual, axis=-1, keepdims=True)
        cross = jnp.dot(-2.0 * residual, cb.T,
                        preferred_element_type=jnp.float32)
        dists = rr + cross + cb2[None, :]
        m = jnp.min(dists, axis=-1, keepdims=True)
        idxf = jnp.min(jnp.where(dists == m, fiota,
                                 jnp.float32(CODEBOOK_SIZE)), axis=-1)
        ohb = (fiota == idxf[:, None]).astype(jnp.bfloat16)
        q = (jnp.dot(ohb, cb_hi, preferred_element_type=jnp.float32)
             + jnp.dot(ohb, cb_mid, preferred_element_type=jnp.float32)
             + jnp.dot(ohb, cb_lo, preferred_element_type=jnp.float32))
        idx_rows.append(idxf.astype(jnp.int32))
        count_rows.append(jnp.dot(ones_row, ohb,
                                  preferred_element_type=jnp.float32)[0])
        q_acc = q_acc + jnp.sum((residual - q) ** 2)
        z_q = z_q + q
        residual = residual - q

    zq_ref[...] = z_q
    codes_ref[...] = jnp.stack(idx_rows, axis=0).T
    qloss_ref[...] = ((1.0 + BETA) / (BATCH * LATENT_DIM)
                      * q_acc).reshape(1, 1)
    p = jnp.stack(count_rows, axis=0) * (1.0 / BATCH)
    usage_ref[...] = (USAGE_REG
                      * jnp.sum(p * jnp.log(p + 1e-10))).reshape(1, 1)


def _dec_kernel(zq_ref, Wd1_ref, bd1_ref, Wd2_ref, bd2_ref, out_ref):
    hd = jnp.maximum(jnp.dot(zq_ref[...], Wd1_ref[...],
                             preferred_element_type=jnp.float32)
                     + bd1_ref[0, :], 0.0)
    r = jnp.dot(hd, Wd2_ref[...],
                preferred_element_type=jnp.float32) + bd2_ref[0, :]
    out_ref[...] = r.reshape(TILE, CHUNK_LEN, FRAME_DIM)


@jax.jit
def kernel(x, We1, be1, We2, be2, Wd1, bd1, Wd2, bd2, codebooks):
    B = x.shape[0]

    z = pl.pallas_call(
        _enc_kernel,
        grid=(NB,),
        in_specs=[
            pl.BlockSpec((TILE, CHUNK_LEN, FRAME_DIM), lambda i: (i, 0, 0)),
            pl.BlockSpec((IN_DIM, HIDDEN), lambda i: (0, 0)),
            pl.BlockSpec((1, HIDDEN), lambda i: (0, 0)),
            pl.BlockSpec((HIDDEN, LATENT_DIM), lambda i: (0, 0)),
            pl.BlockSpec((1, LATENT_DIM), lambda i: (0, 0)),
        ],
        out_specs=pl.BlockSpec((TILE, LATENT_DIM), lambda i: (i, 0)),
        out_shape=jax.ShapeDtypeStruct((B, LATENT_DIM), jnp.float32),
    )(x, We1, be1.reshape(1, HIDDEN), We2, be2.reshape(1, LATENT_DIM))

    z_q, codes, q_loss, usage_loss = pl.pallas_call(
        _vq_kernel,
        out_shape=[
            jax.ShapeDtypeStruct((B, LATENT_DIM), jnp.float32),
            jax.ShapeDtypeStruct((B, LEVELS), jnp.int32),
            jax.ShapeDtypeStruct((1, 1), jnp.float32),
            jax.ShapeDtypeStruct((1, 1), jnp.float32),
        ],
    )(z, codebooks)

    recon = pl.pallas_call(
        _dec_kernel,
        grid=(NB,),
        in_specs=[
            pl.BlockSpec((TILE, LATENT_DIM), lambda i: (i, 0)),
            pl.BlockSpec((LATENT_DIM, HIDDEN), lambda i: (0, 0)),
            pl.BlockSpec((1, HIDDEN), lambda i: (0, 0)),
            pl.BlockSpec((HIDDEN, IN_DIM), lambda i: (0, 0)),
            pl.BlockSpec((1, IN_DIM), lambda i: (0, 0)),
        ],
        out_specs=pl.BlockSpec((TILE, CHUNK_LEN, FRAME_DIM),
                               lambda i: (i, 0, 0)),
        out_shape=jax.ShapeDtypeStruct((B, CHUNK_LEN, FRAME_DIM),
                                       jnp.float32),
    )(z_q, Wd1, bd1.reshape(1, HIDDEN), Wd2, bd2.reshape(1, IN_DIM))

    return recon, codes, q_loss.reshape(()), usage_loss.reshape(()), z_q


# speedup vs baseline: 2.1645x; 1.0623x over previous
import jax
import jax.numpy as jnp
from jax.experimental import pallas as pl
from jax.experimental.pallas import tpu as pltpu

FRAME_DIM = 256
CHUNK_LEN = 32
LATENT_DIM = 256
CODEBOOK_SIZE = 1024
LEVELS = 8
HIDDEN = 1024
BETA = 0.25
USAGE_REG = 0.001
BATCH = 1024

TILE = 256
NB = BATCH // TILE
IN_DIM = CHUNK_LEN * FRAME_DIM


def _enc_kernel(x_ref, We1_ref, be1_ref, We2_ref, be2_ref, z_ref):
    xf = x_ref[...].reshape(TILE, IN_DIM)
    h = jnp.maximum(jnp.dot(xf, We1_ref[...],
                            preferred_element_type=jnp.float32)
                    + be1_ref[0, :], 0.0)
    z_ref[...] = jnp.dot(h, We2_ref[...],
                         preferred_element_type=jnp.float32) + be2_ref[0, :]


def _vq_kernel(z_ref, cb_ref, zq_ref, codes_ref, qloss_ref, usage_ref):
    z = z_ref[...]
    residual = z
    z_q = jnp.zeros_like(z)
    q_acc = jnp.float32(0.0)
    idx_rows = []
    count_rows = []
    fiota = jax.lax.broadcasted_iota(
        jnp.int32, (BATCH, CODEBOOK_SIZE), 1).astype(jnp.float32)
    for l in range(LEVELS):
        cb = cb_ref[l]
        cb_hi = cb.astype(jnp.bfloat16)
        cb_mid = (cb - cb_hi.astype(jnp.float32)).astype(jnp.bfloat16)
        cb2 = jnp.sum(cb * cb, axis=-1)
        rr = jnp.sum(residual * residual, axis=-1, keepdims=True)
        cross = jnp.dot(-2.0 * residual, cb.T,
                        preferred_element_type=jnp.float32)
        dists = rr + cross + cb2[None, :]
        m = jnp.min(dists, axis=-1, keepdims=True)
        idxf = jnp.min(jnp.where(dists == m, fiota,
                                 jnp.float32(CODEBOOK_SIZE)), axis=-1)
        ohb = (fiota == idxf[:, None]).astype(jnp.bfloat16)
        q = (jnp.dot(ohb, cb_hi, preferred_element_type=jnp.float32)
             + jnp.dot(ohb, cb_mid, preferred_element_type=jnp.float32))
        idx_rows.append(idxf.astype(jnp.int32))
        count_rows.append(jnp.sum(ohb.astype(jnp.float32), axis=0))
        q_acc = q_acc + jnp.sum((residual - q) ** 2)
        z_q = z_q + q
        residual = residual - q

    zq_ref[...] = z_q
    codes_ref[...] = jnp.stack(idx_rows, axis=0).T
    qloss_ref[...] = ((1.0 + BETA) / (BATCH * LATENT_DIM)
                      * q_acc).reshape(1, 1)
    p = jnp.stack(count_rows, axis=0) * (1.0 / BATCH)
    usage_ref[...] = (USAGE_REG
                      * jnp.sum(p * jnp.log(p + 1e-10))).reshape(1, 1)


def _dec_kernel(zq_ref, Wd1_ref, bd1_ref, Wd2_ref, bd2_ref, out_ref):
    hd = jnp.maximum(jnp.dot(zq_ref[...], Wd1_ref[...],
                             preferred_element_type=jnp.float32)
                     + bd1_ref[0, :], 0.0)
    r = jnp.dot(hd, Wd2_ref[...],
                preferred_element_type=jnp.float32) + bd2_ref[0, :]
    out_ref[...] = r.reshape(TILE, CHUNK_LEN, FRAME_DIM)


@jax.jit
def kernel(x, We1, be1, We2, be2, Wd1, bd1, Wd2, bd2, codebooks):
    B = x.shape[0]

    z = pl.pallas_call(
        _enc_kernel,
        grid=(NB,),
        in_specs=[
            pl.BlockSpec((TILE, CHUNK_LEN, FRAME_DIM), lambda i: (i, 0, 0)),
            pl.BlockSpec((IN_DIM, HIDDEN), lambda i: (0, 0)),
            pl.BlockSpec((1, HIDDEN), lambda i: (0, 0)),
            pl.BlockSpec((HIDDEN, LATENT_DIM), lambda i: (0, 0)),
            pl.BlockSpec((1, LATENT_DIM), lambda i: (0, 0)),
        ],
        out_specs=pl.BlockSpec((TILE, LATENT_DIM), lambda i: (i, 0)),
        out_shape=jax.ShapeDtypeStruct((B, LATENT_DIM), jnp.float32),
    )(x, We1, be1.reshape(1, HIDDEN), We2, be2.reshape(1, LATENT_DIM))

    z_q, codes, q_loss, usage_loss = pl.pallas_call(
        _vq_kernel,
        out_shape=[
            jax.ShapeDtypeStruct((B, LATENT_DIM), jnp.float32),
            jax.ShapeDtypeStruct((B, LEVELS), jnp.int32),
            jax.ShapeDtypeStruct((1, 1), jnp.float32),
            jax.ShapeDtypeStruct((1, 1), jnp.float32),
        ],
    )(z, codebooks)

    recon = pl.pallas_call(
        _dec_kernel,
        grid=(NB,),
        in_specs=[
            pl.BlockSpec((TILE, LATENT_DIM), lambda i: (i, 0)),
            pl.BlockSpec((LATENT_DIM, HIDDEN), lambda i: (0, 0)),
            pl.BlockSpec((1, HIDDEN), lambda i: (0, 0)),
            pl.BlockSpec((HIDDEN, IN_DIM), lambda i: (0, 0)),
            pl.BlockSpec((1, IN_DIM), lambda i: (0, 0)),
        ],
        out_specs=pl.BlockSpec((TILE, CHUNK_LEN, FRAME_DIM),
                               lambda i: (i, 0, 0)),
        out_shape=jax.ShapeDtypeStruct((B, CHUNK_LEN, FRAME_DIM),
                                       jnp.float32),
    )(z_q, Wd1, bd1.reshape(1, HIDDEN), Wd2, bd2.reshape(1, IN_DIM))

    return recon, codes, q_loss.reshape(()), usage_loss.reshape(()), z_q
